# baseline (speedup 1.0000x reference)
import functools

import jax
import jax.numpy as jnp
from jax import lax
from jax.experimental import pallas as pl
from jax.experimental.pallas import tpu as pltpu
from jax.experimental.pallas import tpu_sc as plsc

L = 16
NSET = 3
CH = 8


@functools.lru_cache(maxsize=None)
def _make_sc_lookup(B, S, D):
    info = plsc.get_sparse_core_info()
    NC, NS = info.num_cores, info.num_subcores
    NW = NC * NS
    N = B * S
    SPW = S // NW
    NG = SPW // CH
    mesh = plsc.VectorSubcoreMesh(core_axis_name="c", subcore_axis_name="s")

    @functools.partial(
        pl.kernel,
        mesh=mesh,
        out_type=jax.ShapeDtypeStruct((N, D), jnp.float32),
        scratch_types=[
            pltpu.VMEM((B * SPW,), jnp.int32),
            *[pltpu.VMEM((CH, D), jnp.float32) for _ in range(2)],
            *[pltpu.VMEM((CH, D), jnp.float32) for _ in range(NSET * B)],
            *[pltpu.SemaphoreType.DMA for _ in range(2)],
            *[pltpu.SemaphoreType.DMA for _ in range(NSET)],
            *[pltpu.SemaphoreType.DMA for _ in range(NSET)],
        ],
    )
    def lookup(tokens_hbm, table_hbm, pos_hbm, out_hbm,
               idx_v, *refs):
        pps = refs[:2]
        sets = [refs[2 + q * B:2 + (q + 1) * B] for q in range(NSET)]
        psems = refs[2 + NSET * B:4 + NSET * B]
        gsems = refs[4 + NSET * B:4 + NSET * B + NSET]
        ssems = refs[4 + NSET * B + NSET:]

        wid = lax.axis_index("s") * NC + lax.axis_index("c")
        s0 = wid * SPW

        def start_pos(g):
            return pltpu.async_copy(
                pos_hbm.at[pl.ds(s0 + g * CH, CH)], pps[g % 2], psems[g % 2])

        pos_cps = {g: start_pos(g) for g in range(min(2, NG))}

        for b in range(B):
            pltpu.sync_copy(tokens_hbm.at[pl.ds(b * S + s0, SPW)],
                            idx_v.at[pl.ds(b * SPW, SPW)])

        def start_gathers(g):
            q = g % NSET
            return [pltpu.async_copy(
                table_hbm.at[idx_v.at[pl.ds(b * SPW + g * CH, CH)]],
                sets[q][b], gsems[q]) for b in range(B)]

        gathers = {g: start_gathers(g) for g in range(min(2, NG))}
        stores = {}

        for g in range(NG):
            q = g % NSET
            pos_cps[g].wait()
            for cp in gathers[g]:
                cp.wait()

            pp = pps[g % 2]
            sbufs = sets[q]

            def row_body(i, carry, _sbufs=sbufs, _pp=pp):
                for j in range(D // L):
                    sl = pl.ds(j * L, L)
                    v = _pp[i, sl]
                    for bb in range(B):
                        plsc.addupdate(_sbufs[bb].at[i, sl], v)
                return carry

            lax.fori_loop(0, CH, row_body, 0)
            stores[g] = [pltpu.async_copy(
                sbufs[b], out_hbm.at[pl.ds(b * S + s0 + g * CH, CH)],
                ssems[q]) for b in range(B)]
            if g + 2 < NG:
                pos_cps[g + 2] = start_pos(g + 2)
                if g >= 1:
                    for cp in stores[g - 1]:
                        cp.wait()
                gathers[g + 2] = start_gathers(g + 2)

        for g in range(max(0, NG - NSET), NG):
            for cp in stores[g]:
                cp.wait()

    return lookup


def kernel(tokens, token_emb, pos_enc):
    B, S = tokens.shape
    D = token_emb.shape[1]
    tok_flat = tokens.reshape(-1).astype(jnp.int32)
    pos2d = pos_enc[0, :S, :]
    out = _make_sc_lookup(B, S, D)(tok_flat, token_emb, pos2d)
    return out.reshape(B, S, D)


# speedup vs baseline: 1.0646x; 1.0646x over previous
import functools

import jax
import jax.numpy as jnp
from jax import lax
from jax.experimental import pallas as pl
from jax.experimental.pallas import tpu as pltpu
from jax.experimental.pallas import tpu_sc as plsc

L = 16
NSET = 3
CH = 8


@functools.lru_cache(maxsize=None)
def _make_sc_lookup(B, S, D):
    info = plsc.get_sparse_core_info()
    NC, NS = info.num_cores, info.num_subcores
    NW = NC * NS
    N = B * S
    SPW = S // NW
    NG = SPW // CH
    mesh = plsc.VectorSubcoreMesh(core_axis_name="c", subcore_axis_name="s")

    @functools.partial(
        pl.kernel,
        mesh=mesh,
        out_type=jax.ShapeDtypeStruct((N, D), jnp.float32),
        scratch_types=[
            pltpu.VMEM((B * SPW,), jnp.int32),
            *[pltpu.VMEM((CH, D), jnp.float32) for _ in range(2)],
            *[pltpu.VMEM((CH, D), jnp.float32) for _ in range(NSET * B)],
            *[pltpu.SemaphoreType.DMA for _ in range(2)],
            *[pltpu.SemaphoreType.DMA for _ in range(NSET)],
            *[pltpu.SemaphoreType.DMA for _ in range(NSET)],
        ],
    )
    def lookup(tokens_hbm, table_hbm, pos_hbm, out_hbm,
               idx_v, *refs):
        pps = refs[:2]
        sets = [refs[2 + q * B:2 + (q + 1) * B] for q in range(NSET)]
        psems = refs[2 + NSET * B:4 + NSET * B]
        gsems = refs[4 + NSET * B:4 + NSET * B + NSET]
        ssems = refs[4 + NSET * B + NSET:]

        wid = lax.axis_index("s") * NC + lax.axis_index("c")
        s0 = wid * SPW

        def start_pos(g):
            return pltpu.async_copy(
                pos_hbm.at[pl.ds(s0 + g * CH, CH)], pps[g % 2], psems[g % 2])

        pos_cps = {g: start_pos(g) for g in range(min(2, NG))}

        for b in range(B):
            pltpu.sync_copy(tokens_hbm.at[pl.ds(b * S + s0, SPW)],
                            idx_v.at[pl.ds(b * SPW, SPW)])

        def start_gathers(g):
            q = g % NSET
            return [pltpu.async_copy(
                table_hbm.at[idx_v.at[pl.ds(b * SPW + g * CH, CH)]],
                sets[q][b], gsems[q]) for b in range(B)]

        gathers = {g: start_gathers(g) for g in range(min(2, NG))}
        stores = {}

        for g in range(NG):
            q = g % NSET
            pos_cps[g].wait()
            for cp in gathers[g]:
                cp.wait()

            pp = pps[g % 2]
            sbufs = sets[q]

            def row_body(i, carry, _sbufs=sbufs, _pp=pp):
                for j in range(D // L):
                    sl = pl.ds(j * L, L)
                    v = _pp[i, sl]
                    for bb in range(B):
                        _sbufs[bb][i, sl] = _sbufs[bb][i, sl] + v
                return carry

            lax.fori_loop(0, CH, row_body, 0)
            stores[g] = [pltpu.async_copy(
                sbufs[b], out_hbm.at[pl.ds(b * S + s0 + g * CH, CH)],
                ssems[q]) for b in range(B)]
            if g + 2 < NG:
                pos_cps[g + 2] = start_pos(g + 2)
                if g >= 1:
                    for cp in stores[g - 1]:
                        cp.wait()
                gathers[g + 2] = start_gathers(g + 2)

        for g in range(max(0, NG - NSET), NG):
            for cp in stores[g]:
                cp.wait()

    return lookup


def kernel(tokens, token_emb, pos_enc):
    B, S = tokens.shape
    D = token_emb.shape[1]
    tok_flat = tokens.reshape(-1).astype(jnp.int32)
    pos2d = pos_enc[0, :S, :]
    out = _make_sc_lookup(B, S, D)(tok_flat, token_emb, pos2d)
    return out.reshape(B, S, D)


# speedup vs baseline: 1.0946x; 1.0281x over previous
import functools

import jax
import jax.numpy as jnp
from jax import lax
from jax.experimental import pallas as pl
from jax.experimental.pallas import tpu as pltpu
from jax.experimental.pallas import tpu_sc as plsc

L = 16
NBUF = 5
LOOKAHEAD = 3
CH = 16


@functools.lru_cache(maxsize=None)
def _make_sc_lookup(B, S, D):
    info = plsc.get_sparse_core_info()
    NC, NS = info.num_cores, info.num_subcores
    NW = NC * NS
    N = B * S
    SPW = S // NW
    CPB = SPW // CH
    NCHT = B * CPB
    mesh = plsc.VectorSubcoreMesh(core_axis_name="c", subcore_axis_name="s")

    @functools.partial(
        pl.kernel,
        mesh=mesh,
        out_type=jax.ShapeDtypeStruct((N, D), jnp.float32),
        scratch_types=[
            pltpu.VMEM((B * SPW,), jnp.int32),
            *[pltpu.VMEM((CH, D), jnp.float32) for _ in range(2)],
            *[pltpu.VMEM((CH, D), jnp.float32) for _ in range(NBUF)],
            pltpu.SemaphoreType.DMA,
            *[pltpu.SemaphoreType.DMA for _ in range(2)],
            *[pltpu.SemaphoreType.DMA for _ in range(NBUF)],
            *[pltpu.SemaphoreType.DMA for _ in range(NBUF)],
        ],
    )
    def lookup(tokens_hbm, table_hbm, pos_hbm, out_hbm,
               idx_v, *refs):
        pps = refs[:2]
        bufs = refs[2:2 + NBUF]
        isem = refs[2 + NBUF]
        psems = refs[3 + NBUF:5 + NBUF]
        gsems = refs[5 + NBUF:5 + 2 * NBUF]
        ssems = refs[5 + 2 * NBUF:]

        wid = lax.axis_index("s") * NC + lax.axis_index("c")
        s0 = wid * SPW

        def start_pos(c):
            return pltpu.async_copy(
                pos_hbm.at[pl.ds(s0 + c * CH, CH)], pps[c % 2], psems[c % 2])

        pos_cps = {c: start_pos(c) for c in range(min(2, CPB))}

        idx_cps = [pltpu.async_copy(tokens_hbm.at[pl.ds(b * S + s0, SPW)],
                                    idx_v.at[pl.ds(b * SPW, SPW)], isem)
                   for b in range(B)]
        for cp in idx_cps:
            cp.wait()

        def start_gather(t):
            c, b = t // B, t % B
            k = t % NBUF
            return pltpu.async_copy(
                table_hbm.at[idx_v.at[pl.ds(b * SPW + c * CH, CH)]],
                bufs[k], gsems[k])

        gathers = {t: start_gather(t) for t in range(min(LOOKAHEAD, NCHT))}
        stores = {}

        for t in range(NCHT):
            c, b = t // B, t % B
            k = t % NBUF
            kp = c % 2
            if b == 0:
                pos_cps[c].wait()
            gathers[t].wait()

            buf = bufs[k]
            pp = pps[kp]

            def row_body(i, carry, _buf=buf, _pp=pp):
                for j in range(D // L):
                    sl = pl.ds(j * L, L)
                    plsc.addupdate(_buf.at[i, sl], _pp[i, sl])
                return carry

            lax.fori_loop(0, CH, row_body, 0)
            out_off = b * S + s0 + c * CH
            stores[t] = pltpu.async_copy(
                buf, out_hbm.at[pl.ds(out_off, CH)], ssems[k])
            if b == B - 1 and c + 2 < CPB:
                pos_cps[c + 2] = start_pos(c + 2)
            if t + LOOKAHEAD < NCHT:
                if t + LOOKAHEAD >= NBUF:
                    stores[t + LOOKAHEAD - NBUF].wait()
                gathers[t + LOOKAHEAD] = start_gather(t + LOOKAHEAD)

        for t in range(max(0, NCHT - NBUF), NCHT):
            stores[t].wait()

    return lookup


def kernel(tokens, token_emb, pos_enc):
    B, S = tokens.shape
    D = token_emb.shape[1]
    tok_flat = tokens.reshape(-1).astype(jnp.int32)
    pos2d = pos_enc[0, :S, :]
    out = _make_sc_lookup(B, S, D)(tok_flat, token_emb, pos2d)
    return out.reshape(B, S, D)
